# Initial kernel scaffold; baseline (speedup 1.0000x reference)
import jax
import jax.numpy as jnp
from jax.experimental import pallas as pl


def kernel(x, edge_index_0, edge_index_1, edge_index_2, edge_vals_0, edge_vals_1, edge_vals_2, W0, W1, W2, fc1_w, fc1_b, fc2_w, fc2_b, fc3_w, fc3_b):
    raise NotImplementedError("write your pallas kernel here")


# speedup vs baseline: 2.9820x; 2.9820x over previous
import functools

import jax
import jax.numpy as jnp
from jax import lax
from jax.experimental import pallas as pl
from jax.experimental.pallas import tpu as pltpu
from jax.experimental.pallas import tpu_sc as plsc

N = 10000
E = 320000
D = 128
K = 3

NC = 2
NS = 16
NW = NC * NS
BLK = 128
NBLK = -(-E // (NW * BLK))
EPW = NBLK * BLK
E_PAD = EPW * NW
N_PAD = 10240
RPT = N_PAD // NS

MB = 1000
FB = 2000


def _mm_body(x_ref, w_ref, o_ref):
    o_ref[0] = jnp.dot(x_ref[...], w_ref[0], preferred_element_type=jnp.float32)


def _matmul(x, wc):
    return pl.pallas_call(
        _mm_body,
        grid=(K, N // MB),
        in_specs=[
            pl.BlockSpec((MB, D), lambda k, b: (b, 0)),
            pl.BlockSpec((1, D, D), lambda k, b: (k, 0, 0)),
        ],
        out_specs=pl.BlockSpec((1, MB, D), lambda k, b: (k, b, 0)),
        out_shape=jax.ShapeDtypeStruct((K, N, D), jnp.float32),
    )(x, wc)


def _sc_body(h_hbm, col_hbm, row_hbm, ev_hbm, zero_hbm, out_hbm,
             colv, rowv, evv, rowsv, acc, sem):
    cid = lax.axis_index("c")
    sid = lax.axis_index("s")
    wid = sid * NC + cid

    for k in range(K):
        pltpu.sync_copy(zero_hbm, acc.at[pl.ds(sid * RPT, RPT)])
        plsc.subcore_barrier()

        def blk_body(b, _):
            base = k * E_PAD + wid * EPW + b * BLK
            pltpu.sync_copy(col_hbm.at[pl.ds(base, BLK)], colv)
            pltpu.sync_copy(row_hbm.at[pl.ds(base, BLK)], rowv)
            pltpu.sync_copy(ev_hbm.at[pl.ds(base, BLK)], evv)
            pltpu.async_copy(h_hbm.at[colv], rowsv, sem).wait()

            def scale(g, _):
                ev16 = evv[pl.ds(g * 16, 16)]
                for j in range(16):
                    s = ev16[j]
                    e = g * 16 + j
                    for c in range(D // 16):
                        sl = pl.ds(c * 16, 16)
                        rowsv[e, sl] = rowsv[e, sl] * s
                return 0

            lax.fori_loop(0, BLK // 16, scale, 0)
            pltpu.sync_copy(rowsv, acc.at[rowv], add=True)
            return 0

        lax.fori_loop(0, NBLK, blk_body, 0)
        plsc.subcore_barrier()
        pltpu.sync_copy(acc.at[pl.ds(sid * RPT, RPT)],
                        out_hbm.at[cid, k, pl.ds(sid * RPT, RPT)])
        plsc.subcore_barrier()


_sc_aggregate = functools.partial(
    pl.kernel,
    out_type=jax.ShapeDtypeStruct((NC, K, N_PAD, D), jnp.float32),
    mesh=plsc.VectorSubcoreMesh(core_axis_name="c", subcore_axis_name="s"),
    scratch_types=[
        pltpu.VMEM((BLK,), jnp.int32),
        pltpu.VMEM((BLK,), jnp.int32),
        pltpu.VMEM((BLK,), jnp.float32),
        pltpu.VMEM((BLK, D), jnp.float32),
        pltpu.VMEM_SHARED((N_PAD, D), jnp.float32),
        pltpu.SemaphoreType.DMA,
    ],
)(_sc_body)


def _fuse_body(p_ref, w1, b1, w2, b2, w3, b3, o_ref):
    t = [jnp.maximum(p_ref[0, i] + p_ref[1, i], 0.0) for i in range(K)]
    u = [jnp.maximum(sum(t[i] * w1[i, j] for i in range(K)) + b1[j], 0.0)
         for j in range(K)]
    v = [jnp.maximum(sum(u[i] * w2[i, j] for i in range(K)) + b2[j], 0.0)
         for j in range(K)]
    o_ref[...] = sum(v[i] * w3[i, 0] for i in range(K)) + b3[0]


def _fuse(p, fc1_w, fc1_b, fc2_w, fc2_b, fc3_w, fc3_b):
    smem = pltpu.SMEM
    return pl.pallas_call(
        _fuse_body,
        grid=(N // FB,),
        in_specs=[
            pl.BlockSpec((NC, K, FB, D), lambda b: (0, 0, b, 0)),
            pl.BlockSpec(memory_space=smem),
            pl.BlockSpec(memory_space=smem),
            pl.BlockSpec(memory_space=smem),
            pl.BlockSpec(memory_space=smem),
            pl.BlockSpec(memory_space=smem),
            pl.BlockSpec(memory_space=smem),
        ],
        out_specs=pl.BlockSpec((FB, D), lambda b: (b, 0)),
        out_shape=jax.ShapeDtypeStruct((N, D), jnp.float32),
    )(p, fc1_w, fc1_b, fc2_w, fc2_b, fc3_w, fc3_b)


def kernel(x, edge_index_0, edge_index_1, edge_index_2,
           edge_vals_0, edge_vals_1, edge_vals_2,
           W0, W1, W2, fc1_w, fc1_b, fc2_w, fc2_b, fc3_w, fc3_b):
    pad = E_PAD - E
    eis = (edge_index_0, edge_index_1, edge_index_2)
    evs = (edge_vals_0, edge_vals_1, edge_vals_2)

    h = _matmul(x, jnp.stack([W0, W1, W2]))
    h2 = h.reshape(K * N, D)

    colp = jnp.concatenate(
        [jnp.pad(eis[k][1], (0, pad)) + k * N for k in range(K)])
    rowp = jnp.concatenate([jnp.pad(eis[k][0], (0, pad)) for k in range(K)])
    evp = jnp.concatenate([jnp.pad(evs[k], (0, pad)) for k in range(K)])
    zero = jnp.zeros((RPT, D), jnp.float32)

    partial = _sc_aggregate(h2, colp, rowp, evp, zero)
    return _fuse(partial, fc1_w, fc1_b, fc2_w, fc2_b, fc3_w, fc3_b)
